# baseline (speedup 1.0000x reference)
import functools

import jax
import jax.numpy as jnp
from jax import lax
from jax.experimental import pallas as pl
from jax.experimental.pallas import tpu as pltpu
from jax.experimental.pallas import tpu_sc as plsc

N = 10000
DI = 128
DO = 128

NC = 2
NS = 16
NW = NC * NS
CHUNK = 128
NSEC = 2

N_PAD = 10112
ROWS_PER_SUB = N_PAD // NS


def _sc_aggregate(x, idx5, zeros):
    sec = idx5.shape[3]
    mesh = plsc.VectorSubcoreMesh(core_axis_name="c", subcore_axis_name="s")

    assert sec % 2 == 0
    @functools.partial(
        pl.kernel,
        out_type=jax.ShapeDtypeStruct((NC, N_PAD, DI), jnp.float32),
        mesh=mesh,
        scratch_types=[
            pltpu.VMEM((2, sec, CHUNK), jnp.int32),
            pltpu.VMEM((2 * CHUNK, DI), jnp.float32),
            pltpu.VMEM_SHARED((N_PAD, DI), jnp.float32),
            pltpu.SemaphoreType.DMA,
            pltpu.SemaphoreType.DMA,
        ],
    )
    def k(x_hbm, idx_hbm, zeros_hbm, out_hbm,
          idx_v, rows_v, agg_sh, sem_a, sem_b):
        cid = lax.axis_index("c")
        sid = lax.axis_index("s")
        wid = cid * NS + sid
        my_rows = pl.ds(sid * ROWS_PER_SUB, ROWS_PER_SUB)
        pltpu.sync_copy(zeros_hbm.at[my_rows], agg_sh.at[my_rows])
        plsc.subcore_barrier()

        bufs = ((pl.ds(0, CHUNK), sem_a), (pl.ds(CHUNK, CHUNK), sem_b))
        for h in range(NSEC):
            pltpu.sync_copy(idx_hbm.at[wid, h], idx_v)
            pltpu.async_copy(x_hbm.at[idx_v.at[0, 0]], rows_v.at[bufs[0][0]],
                             sem_a)

            def body(i, carry):
                for p in range(2):
                    j = 2 * i + p
                    half, sem = bufs[p]
                    nhalf, nsem = bufs[1 - p]

                    @pl.when(j + 1 < sec)
                    def _():
                        pltpu.async_copy(x_hbm.at[idx_v.at[0, j + 1]],
                                         rows_v.at[nhalf], nsem)

                    pltpu.make_async_copy(x_hbm.at[idx_v.at[0, j]],
                                          rows_v.at[half], sem).wait()
                    pltpu.sync_copy(rows_v.at[half], agg_sh.at[idx_v.at[1, j]],
                                    add=True)
                return carry

            lax.fori_loop(0, sec // 2, body, 0, unroll=False)

        plsc.subcore_barrier()
        pltpu.sync_copy(agg_sh.at[my_rows], out_hbm.at[cid].at[my_rows])

    return k(x, idx5, zeros)


def _tc_mlp(x, parts, eps, W1, b1, g1, be1, W2, b2, g2, be2):
    def body(x_ref, p_ref, eps_ref, W1_ref, b1_ref, g1_ref, be1_ref,
             W2_ref, b2_ref, g2_ref, be2_ref, o_ref):
        agg = p_ref[0, :N, :] + p_ref[1, :N, :]
        h = (1.0 + eps_ref[0]) * x_ref[...] + agg
        y = jnp.dot(h, W1_ref[...], preferred_element_type=jnp.float32) + b1_ref[...]
        mu = jnp.mean(y, axis=0, keepdims=True)
        yc = y - mu
        var = jnp.mean(yc * yc, axis=0, keepdims=True)
        y = g1_ref[...] * yc * lax.rsqrt(var + 1e-5) + be1_ref[...]
        y = jnp.maximum(y, 0.0)
        z = jnp.dot(y, W2_ref[...], preferred_element_type=jnp.float32) + b2_ref[...]
        mu2 = jnp.mean(z, axis=0, keepdims=True)
        zc = z - mu2
        var2 = jnp.mean(zc * zc, axis=0, keepdims=True)
        z = g2_ref[...] * zc * lax.rsqrt(var2 + 1e-5) + be2_ref[...]
        o_ref[...] = jnp.maximum(z, 0.0)

    return pl.pallas_call(
        body,
        out_shape=jax.ShapeDtypeStruct((N, DO), jnp.float32),
    )(x, parts, eps, W1, b1, g1, be1, W2, b2, g2, be2)


def kernel(x, edge_index, eps, W1, b1, gamma1, beta1, W2, b2, gamma2, beta2):
    dst = edge_index[0].astype(jnp.int32)
    src = edge_index[1].astype(jnp.int32)
    e = dst.shape[0]
    epw = -(-e // NW)
    cpw = -(-epw // CHUNK)
    cpw += (-cpw) % (2 * NSEC)
    sec = cpw // NSEC
    e_pad = NW * cpw * CHUNK
    pad = e_pad - e
    srcw = jnp.concatenate([src, jnp.zeros((pad,), jnp.int32)]).reshape(
        NW, NSEC, sec, CHUNK)
    dstw = jnp.concatenate([dst, jnp.full((pad,), N, jnp.int32)]).reshape(
        NW, NSEC, sec, CHUNK)
    idx5 = jnp.stack([srcw, dstw], axis=2)
    zeros = jnp.zeros((N_PAD, DI), jnp.float32)
    parts = _sc_aggregate(x, idx5, zeros)
    return _tc_mlp(x, parts, eps, W1, b1, gamma1, beta1, W2, b2, gamma2, beta2)


# speedup vs baseline: 1.0018x; 1.0018x over previous
import functools

import jax
import jax.numpy as jnp
from jax import lax
from jax.experimental import pallas as pl
from jax.experimental.pallas import tpu as pltpu
from jax.experimental.pallas import tpu_sc as plsc

N = 10000
DI = 128
DO = 128

NC = 2
NS = 16
NW = NC * NS
CHUNK = 128
NSEC = 2

N_PAD = 10112
ROWS_PER_SUB = N_PAD // NS


def _sc_aggregate(x, idx5, zeros):
    sec = idx5.shape[3]
    mesh = plsc.VectorSubcoreMesh(core_axis_name="c", subcore_axis_name="s")

    assert sec % 2 == 0
    @functools.partial(
        pl.kernel,
        out_type=jax.ShapeDtypeStruct((NC, N_PAD, DI), jnp.float32),
        mesh=mesh,
        scratch_types=[
            pltpu.VMEM((2, sec, CHUNK), jnp.int32),
            pltpu.VMEM((2 * CHUNK, DI), jnp.float32),
            pltpu.VMEM_SHARED((N_PAD, DI), jnp.float32),
            pltpu.SemaphoreType.DMA,
            pltpu.SemaphoreType.DMA,
        ],
    )
    def k(x_hbm, idx_hbm, zeros_hbm, out_hbm,
          idx_v, rows_v, agg_sh, sem_a, sem_b):
        cid = lax.axis_index("c")
        sid = lax.axis_index("s")
        wid = cid * NS + sid
        my_rows = pl.ds(sid * ROWS_PER_SUB, ROWS_PER_SUB)
        pltpu.sync_copy(zeros_hbm.at[my_rows], agg_sh.at[my_rows])
        plsc.subcore_barrier()

        bufs = ((pl.ds(0, CHUNK), sem_a), (pl.ds(CHUNK, CHUNK), sem_b))
        for h in range(NSEC):
            pltpu.sync_copy(idx_hbm.at[wid, h], idx_v)
            pltpu.async_copy(x_hbm.at[idx_v.at[0, 0]], rows_v.at[bufs[0][0]],
                             sem_a)

            def body(i, carry):
                for p in range(2):
                    j = 2 * i + p
                    half, sem = bufs[p]
                    nhalf, nsem = bufs[1 - p]

                    @pl.when(j + 1 < sec)
                    def _():
                        pltpu.async_copy(x_hbm.at[idx_v.at[0, j + 1]],
                                         rows_v.at[nhalf], nsem)

                    pltpu.make_async_copy(x_hbm.at[idx_v.at[0, j]],
                                          rows_v.at[half], sem).wait()
                    pltpu.sync_copy(rows_v.at[half], agg_sh.at[idx_v.at[1, j]],
                                    add=True)
                return carry

            lax.fori_loop(0, sec // 2, body, 0, unroll=False)

        plsc.subcore_barrier()
        pltpu.sync_copy(agg_sh.at[my_rows], out_hbm.at[cid].at[my_rows])

    return k(x, idx5, zeros)


def _tc_mlp(x, parts, eps, W1, b1, g1, be1, W2, b2, g2, be2):
    def body(x_ref, p_ref, eps_ref, W1_ref, b1_ref, g1_ref, be1_ref,
             W2_ref, b2_ref, g2_ref, be2_ref, o_ref):
        agg = p_ref[0, :N, :] + p_ref[1, :N, :]
        h = (1.0 + eps_ref[0]) * x_ref[...] + agg
        y = jnp.dot(h, W1_ref[...], preferred_element_type=jnp.float32) + b1_ref[...]
        mu = jnp.mean(y, axis=0, keepdims=True)
        yc = y - mu
        var = jnp.mean(yc * yc, axis=0, keepdims=True)
        y = g1_ref[...] * yc * lax.rsqrt(var + 1e-5) + be1_ref[...]
        y = jnp.maximum(y, 0.0)
        z = jnp.dot(y, W2_ref[...], preferred_element_type=jnp.float32) + b2_ref[...]
        mu2 = jnp.mean(z, axis=0, keepdims=True)
        zc = z - mu2
        var2 = jnp.mean(zc * zc, axis=0, keepdims=True)
        z = g2_ref[...] * zc * lax.rsqrt(var2 + 1e-5) + be2_ref[...]
        o_ref[...] = jnp.maximum(z, 0.0)

    return pl.pallas_call(
        body,
        out_shape=jax.ShapeDtypeStruct((N, DO), jnp.float32),
    )(x, parts, eps, W1, b1, g1, be1, W2, b2, g2, be2)


def kernel(x, edge_index, eps, W1, b1, gamma1, beta1, W2, b2, gamma2, beta2):
    dst = edge_index[0].astype(jnp.int32)
    src = edge_index[1].astype(jnp.int32)
    e = dst.shape[0]
    epw = -(-e // NW)
    cpw = -(-epw // CHUNK)
    cpw += (-cpw) % (2 * NSEC)
    sec = cpw // NSEC
    e_pad = NW * cpw * CHUNK
    pad = e_pad - e
    pad_dst = N + jnp.arange(pad, dtype=jnp.int32) % (N_PAD - N)
    srcw = jnp.concatenate([src, jnp.zeros((pad,), jnp.int32)]).reshape(
        NW, NSEC, sec, CHUNK)
    dstw = jnp.concatenate([dst, pad_dst]).reshape(
        NW, NSEC, sec, CHUNK)
    idx5 = jnp.stack([srcw, dstw], axis=2)
    zeros = jnp.zeros((N_PAD, DI), jnp.float32)
    parts = _sc_aggregate(x, idx5, zeros)
    return _tc_mlp(x, parts, eps, W1, b1, gamma1, beta1, W2, b2, gamma2, beta2)


# speedup vs baseline: 3.2963x; 3.2903x over previous
import functools

import jax
import jax.numpy as jnp
from jax import lax
from jax.experimental import pallas as pl
from jax.experimental.pallas import tpu as pltpu
from jax.experimental.pallas import tpu_sc as plsc

N = 10000
DI = 128
DO = 128

NC = 2
NS = 16
NW = NC * NS
CHUNK = 128
NSEC = 2

N_PAD = 10112
ROWS_PER_SUB = N_PAD // NS


def _sc_aggregate(x, idx5, zeros):
    sec = idx5.shape[3]
    mesh = plsc.VectorSubcoreMesh(core_axis_name="c", subcore_axis_name="s")

    assert sec % 2 == 0
    @functools.partial(
        pl.kernel,
        out_type=jax.ShapeDtypeStruct((NC, N_PAD, DI), jnp.float32),
        mesh=mesh,
        scratch_types=[
            pltpu.VMEM((2, sec, CHUNK), jnp.int32),
            pltpu.VMEM((2 * CHUNK, DI), jnp.float32),
            pltpu.VMEM_SHARED((N_PAD, DI), jnp.float32),
            pltpu.SemaphoreType.DMA,
            pltpu.SemaphoreType.DMA,
        ],
    )
    def k(x_hbm, idx_hbm, zeros_hbm, out_hbm,
          idx_v, rows_v, agg_sh, sem_a, sem_b):
        cid = lax.axis_index("c")
        sid = lax.axis_index("s")
        wid = cid * NS + sid
        my_rows = pl.ds(sid * ROWS_PER_SUB, ROWS_PER_SUB)
        pltpu.sync_copy(zeros_hbm.at[my_rows], agg_sh.at[my_rows])
        plsc.subcore_barrier()

        bufs = ((pl.ds(0, CHUNK), sem_a), (pl.ds(CHUNK, CHUNK), sem_b))
        for h in range(NSEC):
            pltpu.sync_copy(idx_hbm.at[wid, h], idx_v)
            pltpu.async_copy(x_hbm.at[idx_v.at[0, 0]], rows_v.at[bufs[0][0]],
                             sem_a)

            def body(i, carry):
                for p in range(2):
                    j = 2 * i + p
                    half, sem = bufs[p]
                    nhalf, nsem = bufs[1 - p]

                    @pl.when(j + 1 < sec)
                    def _():
                        pltpu.async_copy(x_hbm.at[idx_v.at[0, j + 1]],
                                         rows_v.at[nhalf], nsem)

                    pltpu.make_async_copy(x_hbm.at[idx_v.at[0, j]],
                                          rows_v.at[half], sem).wait()
                    pltpu.sync_copy(rows_v.at[half], agg_sh.at[idx_v.at[1, j]],
                                    add=True)
                return carry

            lax.fori_loop(0, sec // 2, body, 0, unroll=False)

        plsc.subcore_barrier()
        pltpu.sync_copy(agg_sh.at[my_rows], out_hbm.at[cid].at[my_rows])

    return k(x, idx5, zeros)


def _tc_mlp(x, parts, eps, W1, b1, g1, be1, W2, b2, g2, be2):
    def body(x_ref, p_ref, eps_ref, W1_ref, b1_ref, g1_ref, be1_ref,
             W2_ref, b2_ref, g2_ref, be2_ref, o_ref):
        agg = p_ref[0, :N, :] + p_ref[1, :N, :]
        h = (1.0 + eps_ref[0]) * x_ref[...] + agg
        y = jnp.dot(h, W1_ref[...], preferred_element_type=jnp.float32) + b1_ref[...]
        mu = jnp.mean(y, axis=0, keepdims=True)
        yc = y - mu
        var = jnp.mean(yc * yc, axis=0, keepdims=True)
        y = g1_ref[...] * yc * lax.rsqrt(var + 1e-5) + be1_ref[...]
        y = jnp.maximum(y, 0.0)
        z = jnp.dot(y, W2_ref[...], preferred_element_type=jnp.float32) + b2_ref[...]
        mu2 = jnp.mean(z, axis=0, keepdims=True)
        zc = z - mu2
        var2 = jnp.mean(zc * zc, axis=0, keepdims=True)
        z = g2_ref[...] * zc * lax.rsqrt(var2 + 1e-5) + be2_ref[...]
        o_ref[...] = jnp.maximum(z, 0.0)

    return pl.pallas_call(
        body,
        out_shape=jax.ShapeDtypeStruct((N, DO), jnp.float32),
    )(x, parts, eps, W1, b1, g1, be1, W2, b2, g2, be2)


def kernel(x, edge_index, eps, W1, b1, gamma1, beta1, W2, b2, gamma2, beta2):
    dst = edge_index[0].astype(jnp.int32)
    src = edge_index[1].astype(jnp.int32)
    e = dst.shape[0]
    epw = -(-e // NW)
    cpw = -(-epw // CHUNK)
    cpw += (-cpw) % (2 * NSEC)
    sec = cpw // NSEC
    e_pad = NW * cpw * CHUNK
    pad = e_pad - e
    pad_dst = N + jnp.arange(pad, dtype=jnp.int32) % (N_PAD - N)
    pad_src = jnp.arange(pad, dtype=jnp.int32) % N
    srcw = jnp.concatenate([src, pad_src]).reshape(
        NW, NSEC, sec, CHUNK)
    dstw = jnp.concatenate([dst, pad_dst]).reshape(
        NW, NSEC, sec, CHUNK)
    idx5 = jnp.stack([srcw, dstw], axis=2)
    zeros = jnp.zeros((N_PAD, DI), jnp.float32)
    parts = _sc_aggregate(x, idx5, zeros)
    return _tc_mlp(x, parts, eps, W1, b1, gamma1, beta1, W2, b2, gamma2, beta2)


# speedup vs baseline: 3.3693x; 1.0222x over previous
import functools

import jax
import jax.numpy as jnp
from jax import lax
from jax.experimental import pallas as pl
from jax.experimental.pallas import tpu as pltpu
from jax.experimental.pallas import tpu_sc as plsc

N = 10000
DI = 128
DO = 128

NC = 2
NS = 16
NW = NC * NS
CHUNK = 128
NSEC = 2

N_PAD = 10112
ROWS_PER_SUB = N_PAD // NS


def _sc_aggregate(x, src4, dst4):
    sec = src4.shape[2]
    mesh = plsc.VectorSubcoreMesh(core_axis_name="c", subcore_axis_name="s")

    assert sec % 2 == 0
    @functools.partial(
        pl.kernel,
        out_type=jax.ShapeDtypeStruct((NC, N_PAD, DI), jnp.float32),
        mesh=mesh,
        scratch_types=[
            pltpu.VMEM((2, sec, CHUNK), jnp.int32),
            pltpu.VMEM((2 * CHUNK, DI), jnp.float32),
            pltpu.VMEM_SHARED((N_PAD, DI), jnp.float32),
            pltpu.SemaphoreType.DMA,
            pltpu.SemaphoreType.DMA,
        ],
    )
    def k(x_hbm, src_hbm, dst_hbm, out_hbm,
          idx_v, rows_v, agg_sh, sem_a, sem_b):
        cid = lax.axis_index("c")
        sid = lax.axis_index("s")
        wid = cid * NS + sid
        my_rows = pl.ds(sid * ROWS_PER_SUB, ROWS_PER_SUB)
        zv = jnp.zeros((16,), jnp.float32)

        def zbody(r, carry):
            for l in range(DI // 16):
                rows_v[r, pl.ds(l * 16, 16)] = zv
            return carry

        lax.fori_loop(0, 2 * CHUNK, zbody, 0, unroll=False)
        base = sid * ROWS_PER_SUB
        for off in range(0, ROWS_PER_SUB, 2 * CHUNK):
            nrows = min(2 * CHUNK, ROWS_PER_SUB - off)
            pltpu.sync_copy(rows_v.at[pl.ds(0, nrows)],
                            agg_sh.at[pl.ds(base + off, nrows)])
        plsc.subcore_barrier()

        bufs = ((pl.ds(0, CHUNK), sem_a), (pl.ds(CHUNK, CHUNK), sem_b))
        for h in range(NSEC):
            pltpu.sync_copy(src_hbm.at[wid, h], idx_v.at[0])
            pltpu.sync_copy(dst_hbm.at[wid, h], idx_v.at[1])
            pltpu.async_copy(x_hbm.at[idx_v.at[0, 0]], rows_v.at[bufs[0][0]],
                             sem_a)

            def body(i, carry):
                for p in range(2):
                    j = 2 * i + p
                    half, sem = bufs[p]
                    nhalf, nsem = bufs[1 - p]

                    @pl.when(j + 1 < sec)
                    def _():
                        pltpu.async_copy(x_hbm.at[idx_v.at[0, j + 1]],
                                         rows_v.at[nhalf], nsem)

                    pltpu.make_async_copy(x_hbm.at[idx_v.at[0, j]],
                                          rows_v.at[half], sem).wait()
                    pltpu.sync_copy(rows_v.at[half], agg_sh.at[idx_v.at[1, j]],
                                    add=True)
                return carry

            lax.fori_loop(0, sec // 2, body, 0, unroll=False)

        plsc.subcore_barrier()
        pltpu.sync_copy(agg_sh.at[my_rows], out_hbm.at[cid].at[my_rows])

    return k(x, src4, dst4)


def _tc_mlp(x, parts, eps, W1, b1, g1, be1, W2, b2, g2, be2):
    def body(x_ref, p_ref, eps_ref, W1_ref, b1_ref, g1_ref, be1_ref,
             W2_ref, b2_ref, g2_ref, be2_ref, o_ref):
        agg = p_ref[0, :N, :] + p_ref[1, :N, :]
        h = (1.0 + eps_ref[0]) * x_ref[...] + agg
        y = jnp.dot(h, W1_ref[...], preferred_element_type=jnp.float32) + b1_ref[...]
        mu = jnp.mean(y, axis=0, keepdims=True)
        yc = y - mu
        var = jnp.mean(yc * yc, axis=0, keepdims=True)
        y = g1_ref[...] * yc * lax.rsqrt(var + 1e-5) + be1_ref[...]
        y = jnp.maximum(y, 0.0)
        z = jnp.dot(y, W2_ref[...], preferred_element_type=jnp.float32) + b2_ref[...]
        mu2 = jnp.mean(z, axis=0, keepdims=True)
        zc = z - mu2
        var2 = jnp.mean(zc * zc, axis=0, keepdims=True)
        z = g2_ref[...] * zc * lax.rsqrt(var2 + 1e-5) + be2_ref[...]
        o_ref[...] = jnp.maximum(z, 0.0)

    return pl.pallas_call(
        body,
        out_shape=jax.ShapeDtypeStruct((N, DO), jnp.float32),
    )(x, parts, eps, W1, b1, g1, be1, W2, b2, g2, be2)


def kernel(x, edge_index, eps, W1, b1, gamma1, beta1, W2, b2, gamma2, beta2):
    dst = edge_index[0].astype(jnp.int32)
    src = edge_index[1].astype(jnp.int32)
    e = dst.shape[0]
    epw = -(-e // NW)
    cpw = -(-epw // CHUNK)
    cpw += (-cpw) % (2 * NSEC)
    sec = cpw // NSEC
    e_pad = NW * cpw * CHUNK
    pad = e_pad - e
    pad_dst = N + jnp.arange(pad, dtype=jnp.int32) % (N_PAD - N)
    pad_src = jnp.arange(pad, dtype=jnp.int32) % N
    src4 = jnp.concatenate([src, pad_src]).reshape(NW, NSEC, sec, CHUNK)
    dst4 = jnp.concatenate([dst, pad_dst]).reshape(NW, NSEC, sec, CHUNK)
    parts = _sc_aggregate(x, src4, dst4)
    return _tc_mlp(x, parts, eps, W1, b1, gamma1, beta1, W2, b2, gamma2, beta2)
